# Initial kernel scaffold; baseline (speedup 1.0000x reference)
import jax
import jax.numpy as jnp
from jax.experimental import pallas as pl


def kernel(pred, y, masks_squeezed):
    raise NotImplementedError("write your pallas kernel here")


# speedup vs baseline: 18.3400x; 18.3400x over previous
import functools

import jax
import jax.numpy as jnp
from jax import lax
from jax.experimental import pallas as pl
from jax.experimental.pallas import tpu as pltpu
from jax.experimental.pallas import tpu_sc as plsc

EPS = 1e-8
L = 16
R_IMGS = 32
HW = 512 * 512
W = 512
ROWS = 512
CHUNK_ROWS = 16
CHUNK = CHUNK_ROWS * W
NCHUNKS = HW // CHUNK
NB1 = 2048
NB3 = 1024
import numpy as np

MIN32 = np.int32(-2147483648)

_mesh = plsc.VectorSubcoreMesh(core_axis_name="c", subcore_axis_name="s")


def _lsr(x, n):
    return lax.shift_right_logical(x, jnp.full_like(x, n))


def _transform(v):
    u = plsc.bitcast(v, jnp.int32)
    return jnp.where(u < 0, jnp.bitwise_xor(u, jnp.int32(-1)),
                     jnp.bitwise_or(u, MIN32))


def _untransform(t):
    bits = jnp.where(t < 0, jnp.bitwise_xor(t, MIN32),
                     jnp.bitwise_xor(t, jnp.int32(-1)))
    return plsc.bitcast(bits, jnp.float32) if hasattr(t, "shape") and t.ndim else lax.bitcast_convert_type(bits, jnp.float32)


def _zero(ref, nwords):
    zeros = jnp.zeros((L,), jnp.int32)

    def body(j, _):
        ref[pl.ds(j * L, L)] = zeros
        return 0

    lax.fori_loop(0, nwords // L, body, 0)


def _merge_lanes(hist, merged, nb):
    def body(j, _):
        acc = hist[pl.ds(j * L, L)]
        for l in range(1, L):
            acc = acc + hist[pl.ds(l * nb + j * L, L)]
        merged[pl.ds(j * L, L)] = acc
        return 0

    lax.fori_loop(0, nb // L, body, 0)


def _scan_select(merged, nb, r):
    iota = lax.iota(jnp.int32, L)

    def body(j, carry):
        run, found, sel, cumbef = carry
        v = merged[pl.ds(j * L, L)]
        cum = plsc.cumsum(v)
        tot = jnp.sum(v)
        below = (run + cum) < r
        lane = jnp.sum(below.astype(jnp.int32))
        s_below = jnp.sum(jnp.where(below, v, 0))
        hit = jnp.logical_and(jnp.logical_not(found), (run + tot) >= r)
        sel = jnp.where(hit, j * L + lane, sel)
        cumbef = jnp.where(hit, run + s_below, cumbef)
        found = jnp.logical_or(found, hit)
        return run + tot, found, sel, cumbef

    _, _, sel, cumbef = lax.fori_loop(
        0, nb // L, body,
        (jnp.int32(0), jnp.bool_(False), jnp.int32(0), jnp.int32(0)))
    return sel, cumbef


def _weighted_below(merged, prefix2, sel3):
    iota = lax.iota(jnp.int32, L)
    base = lax.shift_left(prefix2, jnp.int32(10))

    def body(j, acc):
        d = iota + j * L
        cnt = merged[pl.ds(j * L, L)].astype(jnp.float32)
        val = _untransform(jnp.bitwise_or(jnp.full((L,), base, jnp.int32), d))
        take = d < sel3
        return acc + jnp.sum(jnp.where(take, cnt * val, 0.0))

    return lax.fori_loop(0, NB3 // L, body, jnp.float32(0.0))


def _sum512(ref):
    def body(j, acc):
        return acc + ref[pl.ds(j * L, L)]

    acc = lax.fori_loop(0, ROWS // L, body, jnp.zeros((L,), jnp.float32))
    return jnp.sum(acc)


def _sc_body(p_hbm, y_hbm, m_hbm, out_hbm,
             pbuf, ybuf, mbuf,
             hist_p, hist_y, merged,
             rm_sp, rm_sy, rm_spp, rm_syy, rm_spy, rm_cnt,
             outbuf, sem_p, sem_y, sem_m):
    wid = lax.axis_index("s") * 2 + lax.axis_index("c")
    iota = lax.iota(jnp.int32, L)
    lane_off1 = iota * NB1
    zeros_f = jnp.zeros((L,), jnp.float32)
    ones_i = jnp.ones((L,), jnp.int32)

    def load_chunk(c):
        off = c * CHUNK
        cp = pltpu.async_copy(p_hbm.at[wid, pl.ds(off, CHUNK)], pbuf, sem_p)
        cy = pltpu.async_copy(y_hbm.at[wid, pl.ds(off, CHUNK)], ybuf, sem_y)
        cm = pltpu.async_copy(m_hbm.at[wid, pl.ds(off, CHUNK)], mbuf, sem_m)
        cp.wait(); cy.wait(); cm.wait()

    _zero(hist_p, NB1 * L)
    _zero(hist_y, NB1 * L)

    def p1_chunk(c, _):
        load_chunk(c)

        def row(i, carry):
            vsp, vsy, vspp, vsyy, vspy, vscnt = carry

            def col(j, carry2):
                sp, sy, spp, syy, spy, scnt = carry2
                base = i * W + j * L
                p = pbuf[pl.ds(base, L)]
                yv = ybuf[pl.ds(base, L)]
                m = mbuf[pl.ds(base, L)]
                pm = p * m
                ym = yv * m
                sp += pm
                sy += ym
                spp += pm * p
                syy += ym * yv
                spy += pm * yv
                scnt += m
                valid = m != 0.0
                tp = _transform(p)
                ty = _transform(yv)
                plsc.addupdate_scatter(hist_p, [lane_off1 + _lsr(tp, 21)],
                                       ones_i, mask=valid)
                plsc.addupdate_scatter(hist_y, [lane_off1 + _lsr(ty, 21)],
                                       ones_i, mask=valid)
                return sp, sy, spp, syy, spy, scnt

            sp, sy, spp, syy, spy, scnt = lax.fori_loop(
                0, W // L, col, (zeros_f,) * 6)
            lane = iota == i
            vsp = jnp.where(lane, jnp.sum(sp), vsp)
            vsy = jnp.where(lane, jnp.sum(sy), vsy)
            vspp = jnp.where(lane, jnp.sum(spp), vspp)
            vsyy = jnp.where(lane, jnp.sum(syy), vsyy)
            vspy = jnp.where(lane, jnp.sum(spy), vspy)
            vscnt = jnp.where(lane, jnp.sum(scnt), vscnt)
            return vsp, vsy, vspp, vsyy, vspy, vscnt

        vsp, vsy, vspp, vsyy, vspy, vscnt = lax.fori_loop(
            0, CHUNK_ROWS, row, (zeros_f,) * 6)
        rb = c * CHUNK_ROWS
        rm_sp[pl.ds(rb, L)] = vsp
        rm_sy[pl.ds(rb, L)] = vsy
        rm_spp[pl.ds(rb, L)] = vspp
        rm_syy[pl.ds(rb, L)] = vsyy
        rm_spy[pl.ds(rb, L)] = vspy
        rm_cnt[pl.ds(rb, L)] = vscnt
        return 0

    lax.fori_loop(0, NCHUNKS, p1_chunk, 0)

    cnt_f = _sum512(rm_cnt)
    cnt_i = cnt_f.astype(jnp.int32)
    k = lax.shift_right_arithmetic(jnp.maximum(cnt_i, 1) - 1, jnp.int32(1)) + 1

    _merge_lanes(hist_p, merged, NB1)
    sel1_p, cb1_p = _scan_select(merged, NB1, k)
    _merge_lanes(hist_y, merged, NB1)
    sel1_y, cb1_y = _scan_select(merged, NB1, k)
    r_p = k - cb1_p
    r_y = k - cb1_y

    _zero(hist_p, NB1 * L)
    _zero(hist_y, NB1 * L)
    mask11 = jnp.full((L,), 0x7FF, jnp.int32)

    def p2_chunk(c, _):
        load_chunk(c)

        def vec(v, _):
            base = v * L
            p = pbuf[pl.ds(base, L)]
            yv = ybuf[pl.ds(base, L)]
            m = mbuf[pl.ds(base, L)]
            valid = m != 0.0
            tp = _transform(p)
            ty = _transform(yv)
            win_p = jnp.logical_and(valid, _lsr(tp, 21) == sel1_p)
            win_y = jnp.logical_and(valid, _lsr(ty, 21) == sel1_y)
            dp = jnp.bitwise_and(_lsr(tp, 10), mask11)
            dy = jnp.bitwise_and(_lsr(ty, 10), mask11)
            plsc.addupdate_scatter(hist_p, [lane_off1 + dp], ones_i, mask=win_p)
            plsc.addupdate_scatter(hist_y, [lane_off1 + dy], ones_i, mask=win_y)
            return 0

        lax.fori_loop(0, CHUNK // L, vec, 0)
        return 0

    lax.fori_loop(0, NCHUNKS, p2_chunk, 0)

    _merge_lanes(hist_p, merged, NB1)
    sel2_p, cb2_p = _scan_select(merged, NB1, r_p)
    _merge_lanes(hist_y, merged, NB1)
    sel2_y, cb2_y = _scan_select(merged, NB1, r_y)
    pref2_p = jnp.bitwise_or(lax.shift_left(sel1_p, jnp.int32(11)), sel2_p)
    pref2_y = jnp.bitwise_or(lax.shift_left(sel1_y, jnp.int32(11)), sel2_y)
    r_p = r_p - cb2_p
    r_y = r_y - cb2_y

    _zero(hist_p, NB3 * L)
    _zero(hist_y, NB3 * L)
    lane_off3 = iota * NB3
    mask10 = jnp.full((L,), 0x3FF, jnp.int32)

    def p3_chunk(c, carry):
        sa_p, sa_y = carry
        load_chunk(c)

        def vec(v, carry2):
            ap, ay = carry2
            base = v * L
            p = pbuf[pl.ds(base, L)]
            yv = ybuf[pl.ds(base, L)]
            m = mbuf[pl.ds(base, L)]
            valid = m != 0.0
            tp = _transform(p)
            ty = _transform(yv)
            hp = _lsr(tp, 10)
            hy = _lsr(ty, 10)
            win_p = jnp.logical_and(valid, hp == pref2_p)
            win_y = jnp.logical_and(valid, hy == pref2_y)
            below_p = jnp.logical_and(valid, hp < pref2_p)
            below_y = jnp.logical_and(valid, hy < pref2_y)
            ap += jnp.where(below_p, p, 0.0)
            ay += jnp.where(below_y, yv, 0.0)
            dp = jnp.bitwise_and(tp, mask10)
            dy = jnp.bitwise_and(ty, mask10)
            plsc.addupdate_scatter(hist_p, [lane_off3 + dp], ones_i, mask=win_p)
            plsc.addupdate_scatter(hist_y, [lane_off3 + dy], ones_i, mask=win_y)
            return ap, ay

        return lax.fori_loop(0, CHUNK // L, vec, (sa_p, sa_y))

    sa_p, sa_y = lax.fori_loop(0, NCHUNKS, p3_chunk, (zeros_f, zeros_f))
    S_A_p = jnp.sum(sa_p)
    S_A_y = jnp.sum(sa_y)

    _merge_lanes(hist_p, merged, NB3)
    sel3_p, cb3_p = _scan_select(merged, NB3, r_p)
    w_p = _weighted_below(merged, pref2_p, sel3_p)
    _merge_lanes(hist_y, merged, NB3)
    sel3_y, cb3_y = _scan_select(merged, NB3, r_y)
    w_y = _weighted_below(merged, pref2_y, sel3_y)

    has = jnp.full((L,), cnt_i, jnp.int32) > 0
    n_lt_p = jnp.full((L,), cb1_p + cb2_p + cb3_p, jnp.int32).astype(jnp.float32)
    n_lt_y = jnp.full((L,), cb1_y + cb2_y + cb3_y, jnp.int32).astype(jnp.float32)
    pref3_p = jnp.bitwise_or(lax.shift_left(pref2_p, jnp.int32(10)), sel3_p)
    pref3_y = jnp.bitwise_or(lax.shift_left(pref2_y, jnp.int32(10)), sel3_y)
    med_p = jnp.where(has, _untransform(jnp.full((L,), pref3_p, jnp.int32)), 0.0)
    med_y = jnp.where(has, _untransform(jnp.full((L,), pref3_y, jnp.int32)), 0.0)
    S_p = jnp.full((L,), _sum512(rm_sp), jnp.float32)
    S_y = jnp.full((L,), _sum512(rm_sy), jnp.float32)
    cnt_v = jnp.full((L,), cnt_f, jnp.float32)
    sum_abs_p = S_p - 2.0 * (S_A_p + w_p) + med_p * (2.0 * n_lt_p - cnt_v)
    sum_abs_y = S_y - 2.0 * (S_A_y + w_y) + med_y * (2.0 * n_lt_y - cnt_v)
    safe_cnt = jnp.maximum(cnt_v, 1.0)
    eps_v = jnp.full((L,), EPS, jnp.float32)
    sc_p = jnp.where(has, sum_abs_p / safe_cnt + EPS, eps_v)
    sc_y = jnp.where(has, sum_abs_y / safe_cnt + EPS, eps_v)

    a = 1.0 / sc_p
    b = -1.0 / sc_y
    cc = -med_p * a - med_y * b

    def rows(j, acc):
        rb = j * L
        sp = rm_sp[pl.ds(rb, L)]
        sy = rm_sy[pl.ds(rb, L)]
        spp = rm_spp[pl.ds(rb, L)]
        syy = rm_syy[pl.ds(rb, L)]
        spy = rm_spy[pl.ds(rb, L)]
        cr = rm_cnt[pl.ds(rb, L)]
        rho = (a * a) * spp + (b * b) * syy + (2.0 * a * b) * spy \
            + (2.0 * a * cc) * sp + (2.0 * b * cc) * sy + (cc * cc) * cr
        return acc + rho / jnp.maximum(cr, 1.0)

    acc = lax.fori_loop(0, ROWS // L, rows, zeros_f)
    outbuf[...] = acc
    pltpu.sync_copy(outbuf, out_hbm.at[wid])


_sc_kernel = pl.kernel(
    _sc_body,
    out_type=jax.ShapeDtypeStruct((R_IMGS, L), jnp.float32),
    mesh=_mesh,
    compiler_params=pltpu.CompilerParams(needs_layout_passes=False),
    scratch_types=[
        pltpu.VMEM((CHUNK,), jnp.float32),
        pltpu.VMEM((CHUNK,), jnp.float32),
        pltpu.VMEM((CHUNK,), jnp.float32),
        pltpu.VMEM((NB1 * L,), jnp.int32),
        pltpu.VMEM((NB1 * L,), jnp.int32),
        pltpu.VMEM((NB1,), jnp.int32),
        pltpu.VMEM((ROWS,), jnp.float32),
        pltpu.VMEM((ROWS,), jnp.float32),
        pltpu.VMEM((ROWS,), jnp.float32),
        pltpu.VMEM((ROWS,), jnp.float32),
        pltpu.VMEM((ROWS,), jnp.float32),
        pltpu.VMEM((ROWS,), jnp.float32),
        pltpu.VMEM((L,), jnp.float32),
        pltpu.SemaphoreType.DMA,
        pltpu.SemaphoreType.DMA,
        pltpu.SemaphoreType.DMA,
    ],
)


def _tc_finalize_body(x_ref, o_ref):
    val = jnp.sum(x_ref[...]) * (1.0 / (R_IMGS * ROWS))
    o_ref[...] = jnp.broadcast_to(val, (1, 1))


@jax.jit
def kernel(pred, y, masks_squeezed):
    if pred.ndim == 5 and pred.shape[2] == 1:
        pred = jnp.squeeze(pred, axis=2)
    if y.ndim == 5 and y.shape[2] == 1:
        y = jnp.squeeze(y, axis=2)
    p = pred.reshape(R_IMGS, HW)
    q = y.reshape(R_IMGS, HW)
    m = masks_squeezed.reshape(R_IMGS, HW).astype(jnp.float32)
    partials = _sc_kernel(p, q, m)
    out = pl.pallas_call(
        _tc_finalize_body,
        out_shape=jax.ShapeDtypeStruct((1, 1), jnp.float32),
    )(partials)
    return out[0, 0]


# speedup vs baseline: 22.2632x; 1.2139x over previous
import jax
import jax.numpy as jnp
import numpy as np
from jax import lax
from jax.experimental import pallas as pl
from jax.experimental.pallas import tpu as pltpu
from jax.experimental.pallas import tpu_sc as plsc

EPS = 1e-8
L = 16
R_IMGS = 32
W = 512
ROWS = 512
HW = ROWS * W
CHUNK_ROWS = 16
CHUNK = CHUNK_ROWS * W
NCHUNKS = HW // CHUNK
NB1 = 2048
NB3 = 1024
MIN32 = np.int32(-2147483648)

_mesh = plsc.VectorSubcoreMesh(core_axis_name="c", subcore_axis_name="s")


def _lsr(x, n):
    return lax.shift_right_logical(x, jnp.full_like(x, n))


def _untransform(t):
    bits = jnp.where(t < 0, jnp.bitwise_xor(t, MIN32),
                     jnp.bitwise_xor(t, jnp.int32(-1)))
    return plsc.bitcast(bits, jnp.float32)


def _zero(ref, nwords):
    zeros = jnp.zeros((L,), jnp.int32)

    def body(j, _):
        ref[pl.ds(j * L, L)] = zeros
        return 0

    lax.fori_loop(0, nwords // L, body, 0)


def _merge_lanes(hist, merged, nb):
    def body(j, _):
        acc = hist[pl.ds(j * L, L)]
        for l in range(1, L):
            acc = acc + hist[pl.ds(l * nb + j * L, L)]
        merged[pl.ds(j * L, L)] = acc
        return 0

    lax.fori_loop(0, nb // L, body, 0)


def _scan_select(merged, nb, r):
    def body(j, carry):
        run, found, sel, cumbef = carry
        v = merged[pl.ds(j * L, L)]
        cum = plsc.cumsum(v)
        tot = jnp.sum(v)
        below = (run + cum) < r
        lane = jnp.sum(below.astype(jnp.int32))
        s_below = jnp.sum(jnp.where(below, v, 0))
        hit = jnp.logical_and(jnp.logical_not(found), (run + tot) >= r)
        sel = jnp.where(hit, j * L + lane, sel)
        cumbef = jnp.where(hit, run + s_below, cumbef)
        found = jnp.logical_or(found, hit)
        return run + tot, found, sel, cumbef

    _, _, sel, cumbef = lax.fori_loop(
        0, nb // L, body,
        (jnp.int32(0), jnp.bool_(False), jnp.int32(0), jnp.int32(0)))
    return sel, cumbef


def _weighted_below(merged, prefix2, sel3):
    iota = lax.iota(jnp.int32, L)
    base = lax.shift_left(prefix2, jnp.int32(10))

    def body(j, acc):
        d = iota + j * L
        cnt = merged[pl.ds(j * L, L)].astype(jnp.float32)
        val = _untransform(jnp.bitwise_or(jnp.full((L,), base, jnp.int32), d))
        take = d < sel3
        return acc + jnp.sum(jnp.where(take, cnt * val, 0.0))

    return lax.fori_loop(0, NB3 // L, body, jnp.float32(0.0))


def _sc_body(tp_hbm, ty_hbm, out_hbm,
             pbuf, ybuf, hist_p, hist_y, merged,
             outbuf, sem_p, sem_y):
    wid = lax.axis_index("s") * 2 + lax.axis_index("c")
    iota = lax.iota(jnp.int32, L)
    lane_off1 = iota * NB1
    lane_off3 = iota * NB3
    zeros_f = jnp.zeros((L,), jnp.float32)
    ones_i = jnp.ones((L,), jnp.int32)

    def load_chunk(c):
        rb = c * CHUNK_ROWS
        cp = pltpu.async_copy(tp_hbm.at[wid, pl.ds(rb, CHUNK_ROWS)], pbuf,
                              sem_p)
        cy = pltpu.async_copy(ty_hbm.at[wid, pl.ds(rb, CHUNK_ROWS)], ybuf,
                              sem_y)
        cp.wait(); cy.wait()

    def for_each_vec(fn, carry):
        def rowloop(i, car):
            def grp(j, car2):
                for o in range(4):
                    sl = pl.ds(j * 64 + o * L, L)
                    car2 = fn(pbuf[i, sl], ybuf[i, sl], car2)
                return car2

            return lax.fori_loop(0, W // 64, grp, car)

        return lax.fori_loop(0, CHUNK_ROWS, rowloop, carry)

    _zero(hist_p, NB1 * L)
    _zero(hist_y, NB1 * L)

    def p1_vec(tp, ty, cacc):
        valid = tp != -1
        cacc += jnp.where(valid, 1.0, 0.0)
        plsc.addupdate_scatter(hist_p, [lane_off1 + _lsr(tp, 21)], ones_i,
                               mask=valid)
        plsc.addupdate_scatter(hist_y, [lane_off1 + _lsr(ty, 21)], ones_i,
                               mask=valid)
        return cacc

    def p1_chunk(c, cacc):
        load_chunk(c)
        return for_each_vec(p1_vec, cacc)

    cacc = lax.fori_loop(0, NCHUNKS, p1_chunk, zeros_f)
    cnt_f = jnp.sum(cacc)
    cnt_i = cnt_f.astype(jnp.int32)
    k = lax.shift_right_arithmetic(jnp.maximum(cnt_i, 1) - 1, jnp.int32(1)) + 1

    _merge_lanes(hist_p, merged, NB1)
    sel1_p, cb1_p = _scan_select(merged, NB1, k)
    _merge_lanes(hist_y, merged, NB1)
    sel1_y, cb1_y = _scan_select(merged, NB1, k)
    r_p = k - cb1_p
    r_y = k - cb1_y

    _zero(hist_p, NB1 * L)
    _zero(hist_y, NB1 * L)
    mask11 = jnp.full((L,), 0x7FF, jnp.int32)

    def p2_vec(tp, ty, car):
        win_p = _lsr(tp, 21) == sel1_p
        win_y = _lsr(ty, 21) == sel1_y
        dp = jnp.bitwise_and(_lsr(tp, 10), mask11)
        dy = jnp.bitwise_and(_lsr(ty, 10), mask11)
        plsc.addupdate_scatter(hist_p, [lane_off1 + dp], ones_i, mask=win_p)
        plsc.addupdate_scatter(hist_y, [lane_off1 + dy], ones_i, mask=win_y)
        return car

    def p2_chunk(c, car):
        load_chunk(c)
        return for_each_vec(p2_vec, car)

    lax.fori_loop(0, NCHUNKS, p2_chunk, 0)

    _merge_lanes(hist_p, merged, NB1)
    sel2_p, cb2_p = _scan_select(merged, NB1, r_p)
    _merge_lanes(hist_y, merged, NB1)
    sel2_y, cb2_y = _scan_select(merged, NB1, r_y)
    pref2_p = jnp.bitwise_or(lax.shift_left(sel1_p, jnp.int32(11)), sel2_p)
    pref2_y = jnp.bitwise_or(lax.shift_left(sel1_y, jnp.int32(11)), sel2_y)
    r_p = r_p - cb2_p
    r_y = r_y - cb2_y

    _zero(hist_p, NB3 * L)
    _zero(hist_y, NB3 * L)
    mask10 = jnp.full((L,), 0x3FF, jnp.int32)

    def p3_vec(tp, ty, car):
        ap, ay = car
        hp = _lsr(tp, 10)
        hy = _lsr(ty, 10)
        ap += jnp.where(hp < pref2_p, _untransform(tp), 0.0)
        ay += jnp.where(hy < pref2_y, _untransform(ty), 0.0)
        dp = jnp.bitwise_and(tp, mask10)
        dy = jnp.bitwise_and(ty, mask10)
        plsc.addupdate_scatter(hist_p, [lane_off3 + dp], ones_i,
                               mask=hp == pref2_p)
        plsc.addupdate_scatter(hist_y, [lane_off3 + dy], ones_i,
                               mask=hy == pref2_y)
        return ap, ay

    def p3_chunk(c, car):
        load_chunk(c)
        return for_each_vec(p3_vec, car)

    sa_p, sa_y = lax.fori_loop(0, NCHUNKS, p3_chunk, (zeros_f, zeros_f))
    S_A_p = jnp.sum(sa_p)
    S_A_y = jnp.sum(sa_y)

    _merge_lanes(hist_p, merged, NB3)
    sel3_p, cb3_p = _scan_select(merged, NB3, r_p)
    w_p = _weighted_below(merged, pref2_p, sel3_p)
    _merge_lanes(hist_y, merged, NB3)
    sel3_y, cb3_y = _scan_select(merged, NB3, r_y)
    w_y = _weighted_below(merged, pref2_y, sel3_y)

    pref3_p = jnp.bitwise_or(lax.shift_left(pref2_p, jnp.int32(10)), sel3_p)
    pref3_y = jnp.bitwise_or(lax.shift_left(pref2_y, jnp.int32(10)), sel3_y)
    med_p = _untransform(jnp.full((L,), pref3_p, jnp.int32))
    med_y = _untransform(jnp.full((L,), pref3_y, jnp.int32))
    stats = jnp.where(iota == 0, med_p, zeros_f)
    stats = jnp.where(iota == 1, med_y, stats)
    stats = jnp.where(iota == 2, S_A_p + w_p, stats)
    stats = jnp.where(iota == 3, S_A_y + w_y, stats)
    n_lt_p = jnp.full((L,), cb1_p + cb2_p + cb3_p, jnp.int32)
    n_lt_y = jnp.full((L,), cb1_y + cb2_y + cb3_y, jnp.int32)
    stats = jnp.where(iota == 4, n_lt_p.astype(jnp.float32), stats)
    stats = jnp.where(iota == 5, n_lt_y.astype(jnp.float32), stats)
    stats = jnp.where(iota == 6, cnt_f, stats)
    outbuf[...] = stats
    pltpu.sync_copy(outbuf, out_hbm.at[wid])


_sc_kernel = pl.kernel(
    _sc_body,
    out_type=jax.ShapeDtypeStruct((R_IMGS, L), jnp.float32),
    mesh=_mesh,
    compiler_params=pltpu.CompilerParams(needs_layout_passes=False),
    scratch_types=[
        pltpu.VMEM((CHUNK_ROWS, W), jnp.int32),
        pltpu.VMEM((CHUNK_ROWS, W), jnp.int32),
        pltpu.VMEM((NB1 * L,), jnp.int32),
        pltpu.VMEM((NB1 * L,), jnp.int32),
        pltpu.VMEM((NB1,), jnp.int32),
        pltpu.VMEM((L,), jnp.float32),
        pltpu.SemaphoreType.DMA,
        pltpu.SemaphoreType.DMA,
    ],
)


def _tc_prep_body(p_ref, y_ref, m_ref, tp_ref, ty_ref, mom_ref):
    p = p_ref[0]
    yv = y_ref[0]
    m = m_ref[0].astype(jnp.float32)
    valid = m != 0.0

    def transform(x):
        u = lax.bitcast_convert_type(x, jnp.int32)
        t = jnp.where(u < 0, jnp.bitwise_xor(u, jnp.int32(-1)),
                      jnp.bitwise_or(u, MIN32))
        return jnp.where(valid, t, jnp.int32(-1))

    tp_ref[0] = transform(p)
    ty_ref[0] = transform(yv)
    pm = p * m
    ym = yv * m
    mom_ref[0, 0, :] = jnp.sum(pm, axis=-1)
    mom_ref[0, 1, :] = jnp.sum(ym, axis=-1)
    mom_ref[0, 2, :] = jnp.sum(pm * p, axis=-1)
    mom_ref[0, 3, :] = jnp.sum(ym * yv, axis=-1)
    mom_ref[0, 4, :] = jnp.sum(pm * yv, axis=-1)
    mom_ref[0, 5, :] = jnp.sum(m, axis=-1)
    mom_ref[0, 6, :] = jnp.zeros((W,), jnp.float32)
    mom_ref[0, 7, :] = jnp.zeros((W,), jnp.float32)


def _tc_combine_body(mom_ref, st_ref, o_ref):
    st = st_ref[...]
    med_p = st[:, 0:1]
    med_y = st[:, 1:2]
    SL_p = st[:, 2:3]
    SL_y = st[:, 3:4]
    n_lt_p = st[:, 4:5]
    n_lt_y = st[:, 5:6]
    cnt = st[:, 6:7]
    has = cnt > 0.0
    med_p = jnp.where(has, med_p, 0.0)
    med_y = jnp.where(has, med_y, 0.0)
    S_p = jnp.sum(mom_ref[:, 0, :], axis=-1, keepdims=True)
    S_y = jnp.sum(mom_ref[:, 1, :], axis=-1, keepdims=True)
    safe_cnt = jnp.maximum(cnt, 1.0)
    sum_abs_p = S_p - 2.0 * SL_p + med_p * (2.0 * n_lt_p - cnt)
    sum_abs_y = S_y - 2.0 * SL_y + med_y * (2.0 * n_lt_y - cnt)
    sc_p = jnp.where(has, sum_abs_p / safe_cnt + EPS, EPS)
    sc_y = jnp.where(has, sum_abs_y / safe_cnt + EPS, EPS)
    a = 1.0 / sc_p
    b = -1.0 / sc_y
    cc = -med_p * a - med_y * b
    rho = (a * a) * mom_ref[:, 2, :] + (b * b) * mom_ref[:, 3, :] \
        + (2.0 * a * b) * mom_ref[:, 4, :] + (2.0 * a * cc) * mom_ref[:, 0, :] \
        + (2.0 * b * cc) * mom_ref[:, 1, :] + (cc * cc) * mom_ref[:, 5, :]
    per_row = rho / jnp.maximum(mom_ref[:, 5, :], 1.0)
    val = jnp.sum(per_row) * (1.0 / (R_IMGS * ROWS))
    o_ref[...] = jnp.broadcast_to(val, (1, 1))


@jax.jit
def kernel(pred, y, masks_squeezed):
    if pred.ndim == 5 and pred.shape[2] == 1:
        pred = jnp.squeeze(pred, axis=2)
    if y.ndim == 5 and y.shape[2] == 1:
        y = jnp.squeeze(y, axis=2)
    p = pred.reshape(R_IMGS, ROWS, W)
    q = y.reshape(R_IMGS, ROWS, W)
    m = masks_squeezed.reshape(R_IMGS, ROWS, W).astype(jnp.float32)
    img_spec = pl.BlockSpec((1, ROWS, W), lambda i: (i, 0, 0))
    tp, ty, mom = pl.pallas_call(
        _tc_prep_body,
        grid=(R_IMGS,),
        in_specs=[img_spec, img_spec, img_spec],
        out_specs=[img_spec, img_spec,
                   pl.BlockSpec((1, 8, W), lambda i: (i, 0, 0))],
        out_shape=[
            jax.ShapeDtypeStruct((R_IMGS, ROWS, W), jnp.int32),
            jax.ShapeDtypeStruct((R_IMGS, ROWS, W), jnp.int32),
            jax.ShapeDtypeStruct((R_IMGS, 8, W), jnp.float32),
        ],
    )(p, q, m)
    stats = _sc_kernel(tp, ty)
    out = pl.pallas_call(
        _tc_combine_body,
        out_shape=jax.ShapeDtypeStruct((1, 1), jnp.float32),
    )(mom, stats)
    return out[0, 0]


# speedup vs baseline: 25.8857x; 1.1627x over previous
import jax
import jax.numpy as jnp
import numpy as np
from jax import lax
from jax.experimental import pallas as pl
from jax.experimental.pallas import tpu as pltpu
from jax.experimental.pallas import tpu_sc as plsc

EPS = 1e-8
L = 16
R_IMGS = 32
W = 512
ROWS = 512
HW = ROWS * W
CHUNK_ROWS = 16
CHUNK = CHUNK_ROWS * W
NCHUNKS = HW // CHUNK
NB1 = 2048
NB3 = 1024
MIN32 = np.int32(-2147483648)

_mesh = plsc.VectorSubcoreMesh(core_axis_name="c", subcore_axis_name="s")


def _lsr(x, n):
    return lax.shift_right_logical(x, jnp.full_like(x, n))


def _untransform(t):
    bits = jnp.where(t < 0, jnp.bitwise_xor(t, MIN32),
                     jnp.bitwise_xor(t, jnp.int32(-1)))
    return plsc.bitcast(bits, jnp.float32)


def _zero(ref, nwords):
    zeros = jnp.zeros((L,), jnp.int32)

    def body(j, _):
        ref[pl.ds(j * L, L)] = zeros
        return 0

    lax.fori_loop(0, nwords // L, body, 0)


def _scan_vec(ref, nvals, r, run0):
    def body(j, carry):
        run, found, sel, cumbef = carry
        v = ref[pl.ds(j * L, L)]
        cum = plsc.cumsum(v)
        tot = jnp.sum(v)
        below = (run + cum) < r
        lane = jnp.sum(below.astype(jnp.int32))
        s_below = jnp.sum(jnp.where(below, v, 0))
        hit = jnp.logical_and(jnp.logical_not(found), (run + tot) >= r)
        sel = jnp.where(hit, j * L + lane, sel)
        cumbef = jnp.where(hit, run + s_below, cumbef)
        found = jnp.logical_or(found, hit)
        return run + tot, found, sel, cumbef

    _, _, sel, cumbef = lax.fori_loop(
        0, nvals // L, body,
        (run0, jnp.bool_(False), jnp.int32(0), jnp.int32(0)))
    return sel, cumbef


def _scan_select(hist, gsums, nb, r):
    ngrp = nb // L
    iota = lax.iota(jnp.int32, L)

    def g_body(g, _):
        acc = hist[pl.ds(g * L * L, L)]
        for t in range(1, L):
            acc = acc + hist[pl.ds(g * L * L + t * L, L)]
        s = jnp.sum(acc)
        plsc.store_scatter(gsums, [jnp.full((L,), g, jnp.int32)],
                           jnp.full((L,), s, jnp.int32), mask=iota == 0)
        return 0

    lax.fori_loop(0, ngrp, g_body, 0)
    g_sel, g_cumbef = _scan_vec(gsums, ngrp, r, jnp.int32(0))

    def t_body(t, carry):
        run, found, sel, cumbef = carry
        s = jnp.sum(hist[pl.ds((g_sel * L + t) * L, L)])
        hit = jnp.logical_and(jnp.logical_not(found), (run + s) >= r)
        sel = jnp.where(hit, g_sel * L + t, sel)
        cumbef = jnp.where(hit, run, cumbef)
        found = jnp.logical_or(found, hit)
        return run + s, found, sel, cumbef

    _, _, sel, cumbef = lax.fori_loop(
        0, L, t_body,
        (g_cumbef, jnp.bool_(False), jnp.int32(0), jnp.int32(0)))
    return sel, cumbef


def _weighted_below(hist, prefix2, sel3):
    base = lax.shift_left(prefix2, jnp.int32(10))

    def body(d, acc):
        v = hist[pl.ds(d * L, L)].astype(jnp.float32)
        val = _untransform(jnp.full((L,), jnp.bitwise_or(base, d), jnp.int32))
        return acc + jnp.where(d < sel3, v * val, 0.0)

    acc = lax.fori_loop(0, NB3, body, jnp.zeros((L,), jnp.float32))
    return jnp.sum(acc)


def _sc_body(tp_hbm, ty_hbm, out_hbm,
             pbuf0, ybuf0, pbuf1, ybuf1, hist_p, hist_y, merged,
             outbuf, sem_p0, sem_y0, sem_p1, sem_y1):
    wid = lax.axis_index("s") * 2 + lax.axis_index("c")
    iota = lax.iota(jnp.int32, L)
    zeros_f = jnp.zeros((L,), jnp.float32)
    ones_i = jnp.ones((L,), jnp.int32)

    def didx(d):
        return lax.shift_left(d, jnp.full_like(d, 4)) + iota
    slots = ((pbuf0, ybuf0, sem_p0, sem_y0), (pbuf1, ybuf1, sem_p1, sem_y1))

    def prefetch(c, s):
        bp, by, sp, sy = slots[s]
        rb = c * CHUNK_ROWS
        pltpu.async_copy(tp_hbm.at[wid, pl.ds(rb, CHUNK_ROWS)], bp, sp)
        pltpu.async_copy(ty_hbm.at[wid, pl.ds(rb, CHUNK_ROWS)], by, sy)

    def wait_slot(c, s):
        bp, by, sp, sy = slots[s]
        rb = c * CHUNK_ROWS
        pltpu.make_async_copy(tp_hbm.at[wid, pl.ds(rb, CHUNK_ROWS)], bp,
                              sp).wait()
        pltpu.make_async_copy(ty_hbm.at[wid, pl.ds(rb, CHUNK_ROWS)], by,
                              sy).wait()

    def for_each_vec(s, fn, carry):
        bp, by = slots[s][0], slots[s][1]

        def rowloop(i, car):
            def grp(j, car2):
                for o in range(4):
                    sl = pl.ds(j * 64 + o * L, L)
                    car2 = fn(bp[i, sl], by[i, sl], car2)
                return car2

            return lax.fori_loop(0, W // 64, grp, car)

        return lax.fori_loop(0, CHUNK_ROWS, rowloop, carry)

    def run_pass(fn, carry):
        prefetch(0, 0)

        def pair(i, car):
            c0 = 2 * i
            c1 = c0 + 1
            prefetch(c1, 1)
            wait_slot(c0, 0)
            car = for_each_vec(0, fn, car)
            prefetch((c1 + 1) & (NCHUNKS - 1), 0)
            wait_slot(c1, 1)
            car = for_each_vec(1, fn, car)
            return car

        carry = lax.fori_loop(0, NCHUNKS // 2, pair, carry)
        wait_slot(0, 0)
        return carry

    _zero(hist_p, NB1 * L)
    _zero(hist_y, NB1 * L)

    def p1_vec(tp, ty, cacc):
        valid = tp != -1
        cacc += jnp.where(valid, 1.0, 0.0)
        plsc.addupdate_scatter(hist_p, [didx(_lsr(tp, 21))], ones_i,
                               mask=valid)
        plsc.addupdate_scatter(hist_y, [didx(_lsr(ty, 21))], ones_i,
                               mask=valid)
        return cacc

    cacc = run_pass(p1_vec, zeros_f)
    cnt_f = jnp.sum(cacc)
    cnt_i = cnt_f.astype(jnp.int32)
    k = lax.shift_right_arithmetic(jnp.maximum(cnt_i, 1) - 1, jnp.int32(1)) + 1

    sel1_p, cb1_p = _scan_select(hist_p, merged, NB1, k)
    sel1_y, cb1_y = _scan_select(hist_y, merged, NB1, k)
    r_p = k - cb1_p
    r_y = k - cb1_y

    _zero(hist_p, NB1 * L)
    _zero(hist_y, NB1 * L)
    mask11 = jnp.full((L,), 0x7FF, jnp.int32)

    def p2_vec(tp, ty, car):
        win_p = _lsr(tp, 21) == sel1_p
        win_y = _lsr(ty, 21) == sel1_y
        dp = jnp.bitwise_and(_lsr(tp, 10), mask11)
        dy = jnp.bitwise_and(_lsr(ty, 10), mask11)
        plsc.addupdate_scatter(hist_p, [didx(dp)], ones_i, mask=win_p)
        plsc.addupdate_scatter(hist_y, [didx(dy)], ones_i, mask=win_y)
        return car

    run_pass(p2_vec, jnp.int32(0))

    sel2_p, cb2_p = _scan_select(hist_p, merged, NB1, r_p)
    sel2_y, cb2_y = _scan_select(hist_y, merged, NB1, r_y)
    pref2_p = jnp.bitwise_or(lax.shift_left(sel1_p, jnp.int32(11)), sel2_p)
    pref2_y = jnp.bitwise_or(lax.shift_left(sel1_y, jnp.int32(11)), sel2_y)
    r_p = r_p - cb2_p
    r_y = r_y - cb2_y

    _zero(hist_p, NB3 * L)
    _zero(hist_y, NB3 * L)
    mask10 = jnp.full((L,), 0x3FF, jnp.int32)

    def p3_vec(tp, ty, car):
        ap, ay = car
        hp = _lsr(tp, 10)
        hy = _lsr(ty, 10)
        ap += jnp.where(hp < pref2_p, _untransform(tp), 0.0)
        ay += jnp.where(hy < pref2_y, _untransform(ty), 0.0)
        dp = jnp.bitwise_and(tp, mask10)
        dy = jnp.bitwise_and(ty, mask10)
        plsc.addupdate_scatter(hist_p, [didx(dp)], ones_i, mask=hp == pref2_p)
        plsc.addupdate_scatter(hist_y, [didx(dy)], ones_i, mask=hy == pref2_y)
        return ap, ay

    sa_p, sa_y = run_pass(p3_vec, (zeros_f, zeros_f))
    S_A_p = jnp.sum(sa_p)
    S_A_y = jnp.sum(sa_y)

    sel3_p, cb3_p = _scan_select(hist_p, merged, NB3, r_p)
    w_p = _weighted_below(hist_p, pref2_p, sel3_p)
    sel3_y, cb3_y = _scan_select(hist_y, merged, NB3, r_y)
    w_y = _weighted_below(hist_y, pref2_y, sel3_y)

    pref3_p = jnp.bitwise_or(lax.shift_left(pref2_p, jnp.int32(10)), sel3_p)
    pref3_y = jnp.bitwise_or(lax.shift_left(pref2_y, jnp.int32(10)), sel3_y)
    med_p = _untransform(jnp.full((L,), pref3_p, jnp.int32))
    med_y = _untransform(jnp.full((L,), pref3_y, jnp.int32))
    stats = jnp.where(iota == 0, med_p, zeros_f)
    stats = jnp.where(iota == 1, med_y, stats)
    stats = jnp.where(iota == 2, S_A_p + w_p, stats)
    stats = jnp.where(iota == 3, S_A_y + w_y, stats)
    n_lt_p = jnp.full((L,), cb1_p + cb2_p + cb3_p, jnp.int32)
    n_lt_y = jnp.full((L,), cb1_y + cb2_y + cb3_y, jnp.int32)
    stats = jnp.where(iota == 4, n_lt_p.astype(jnp.float32), stats)
    stats = jnp.where(iota == 5, n_lt_y.astype(jnp.float32), stats)
    stats = jnp.where(iota == 6, cnt_f, stats)
    outbuf[...] = stats
    pltpu.sync_copy(outbuf, out_hbm.at[wid])


_sc_kernel = pl.kernel(
    _sc_body,
    out_type=jax.ShapeDtypeStruct((R_IMGS, L), jnp.float32),
    mesh=_mesh,
    compiler_params=pltpu.CompilerParams(needs_layout_passes=False),
    scratch_types=[
        pltpu.VMEM((CHUNK_ROWS, W), jnp.int32),
        pltpu.VMEM((CHUNK_ROWS, W), jnp.int32),
        pltpu.VMEM((CHUNK_ROWS, W), jnp.int32),
        pltpu.VMEM((CHUNK_ROWS, W), jnp.int32),
        pltpu.VMEM((NB1 * L,), jnp.int32),
        pltpu.VMEM((NB1 * L,), jnp.int32),
        pltpu.VMEM((NB1,), jnp.int32),
        pltpu.VMEM((L,), jnp.float32),
        pltpu.SemaphoreType.DMA,
        pltpu.SemaphoreType.DMA,
        pltpu.SemaphoreType.DMA,
        pltpu.SemaphoreType.DMA,
    ],
)


def _tc_prep_body(p_ref, y_ref, m_ref, tp_ref, ty_ref, mom_ref):
    p = p_ref[0]
    yv = y_ref[0]
    m = m_ref[0].astype(jnp.float32)
    valid = m != 0.0

    def transform(x):
        u = lax.bitcast_convert_type(x, jnp.int32)
        t = jnp.where(u < 0, jnp.bitwise_xor(u, jnp.int32(-1)),
                      jnp.bitwise_or(u, MIN32))
        return jnp.where(valid, t, jnp.int32(-1))

    tp_ref[0] = transform(p)
    ty_ref[0] = transform(yv)
    pm = p * m
    ym = yv * m
    mom_ref[0, 0, :] = jnp.sum(pm, axis=-1)
    mom_ref[0, 1, :] = jnp.sum(ym, axis=-1)
    mom_ref[0, 2, :] = jnp.sum(pm * p, axis=-1)
    mom_ref[0, 3, :] = jnp.sum(ym * yv, axis=-1)
    mom_ref[0, 4, :] = jnp.sum(pm * yv, axis=-1)
    mom_ref[0, 5, :] = jnp.sum(m, axis=-1)
    mom_ref[0, 6, :] = jnp.zeros((W,), jnp.float32)
    mom_ref[0, 7, :] = jnp.zeros((W,), jnp.float32)


def _tc_combine_body(mom_ref, st_ref, o_ref):
    st = st_ref[...]
    med_p = st[:, 0:1]
    med_y = st[:, 1:2]
    SL_p = st[:, 2:3]
    SL_y = st[:, 3:4]
    n_lt_p = st[:, 4:5]
    n_lt_y = st[:, 5:6]
    cnt = st[:, 6:7]
    has = cnt > 0.0
    med_p = jnp.where(has, med_p, 0.0)
    med_y = jnp.where(has, med_y, 0.0)
    S_p = jnp.sum(mom_ref[:, 0, :], axis=-1, keepdims=True)
    S_y = jnp.sum(mom_ref[:, 1, :], axis=-1, keepdims=True)
    safe_cnt = jnp.maximum(cnt, 1.0)
    sum_abs_p = S_p - 2.0 * SL_p + med_p * (2.0 * n_lt_p - cnt)
    sum_abs_y = S_y - 2.0 * SL_y + med_y * (2.0 * n_lt_y - cnt)
    sc_p = jnp.where(has, sum_abs_p / safe_cnt + EPS, EPS)
    sc_y = jnp.where(has, sum_abs_y / safe_cnt + EPS, EPS)
    a = 1.0 / sc_p
    b = -1.0 / sc_y
    cc = -med_p * a - med_y * b
    rho = (a * a) * mom_ref[:, 2, :] + (b * b) * mom_ref[:, 3, :] \
        + (2.0 * a * b) * mom_ref[:, 4, :] + (2.0 * a * cc) * mom_ref[:, 0, :] \
        + (2.0 * b * cc) * mom_ref[:, 1, :] + (cc * cc) * mom_ref[:, 5, :]
    per_row = rho / jnp.maximum(mom_ref[:, 5, :], 1.0)
    val = jnp.sum(per_row) * (1.0 / (R_IMGS * ROWS))
    o_ref[...] = jnp.broadcast_to(val, (1, 1))


@jax.jit
def kernel(pred, y, masks_squeezed):
    if pred.ndim == 5 and pred.shape[2] == 1:
        pred = jnp.squeeze(pred, axis=2)
    if y.ndim == 5 and y.shape[2] == 1:
        y = jnp.squeeze(y, axis=2)
    p = pred.reshape(R_IMGS, ROWS, W)
    q = y.reshape(R_IMGS, ROWS, W)
    m = masks_squeezed.reshape(R_IMGS, ROWS, W).astype(jnp.float32)
    img_spec = pl.BlockSpec((1, ROWS, W), lambda i: (i, 0, 0))
    tp, ty, mom = pl.pallas_call(
        _tc_prep_body,
        grid=(R_IMGS,),
        in_specs=[img_spec, img_spec, img_spec],
        out_specs=[img_spec, img_spec,
                   pl.BlockSpec((1, 8, W), lambda i: (i, 0, 0))],
        out_shape=[
            jax.ShapeDtypeStruct((R_IMGS, ROWS, W), jnp.int32),
            jax.ShapeDtypeStruct((R_IMGS, ROWS, W), jnp.int32),
            jax.ShapeDtypeStruct((R_IMGS, 8, W), jnp.float32),
        ],
    )(p, q, m)
    stats = _sc_kernel(tp, ty)
    out = pl.pallas_call(
        _tc_combine_body,
        out_shape=jax.ShapeDtypeStruct((1, 1), jnp.float32),
    )(mom, stats)
    return out[0, 0]


# speedup vs baseline: 44.6556x; 1.7251x over previous
import jax
import jax.numpy as jnp
import numpy as np
from jax import lax
from jax.experimental import pallas as pl
from jax.experimental.pallas import tpu as pltpu
from jax.experimental.pallas import tpu_sc as plsc

EPS = 1e-8
L = 16
R_IMGS = 32
W = 512
ROWS = 512
HW = ROWS * W
CHUNK_ROWS = 16
CHUNK = CHUNK_ROWS * W
NCHUNKS = HW // CHUNK
NB1 = 2048
NB3 = 1024
MIN32 = np.int32(-2147483648)

_mesh = plsc.VectorSubcoreMesh(core_axis_name="c", subcore_axis_name="s")


def _lsr(x, n):
    return lax.shift_right_logical(x, jnp.full_like(x, n))


def _untransform(t):
    bits = jnp.where(t < 0, jnp.bitwise_xor(t, MIN32),
                     jnp.bitwise_xor(t, jnp.int32(-1)))
    return plsc.bitcast(bits, jnp.float32)


def _zero(ref, nwords):
    zeros = jnp.zeros((L,), jnp.int32)

    def body(j, _):
        ref[pl.ds(j * L, L)] = zeros
        return 0

    lax.fori_loop(0, nwords // L, body, 0)


def _scan_vec(ref, nvals, r, run0):
    def body(j, carry):
        run, found, sel, cumbef = carry
        v = ref[pl.ds(j * L, L)]
        cum = plsc.cumsum(v)
        tot = jnp.sum(v)
        below = (run + cum) < r
        lane = jnp.sum(below.astype(jnp.int32))
        s_below = jnp.sum(jnp.where(below, v, 0))
        hit = jnp.logical_and(jnp.logical_not(found), (run + tot) >= r)
        sel = jnp.where(hit, j * L + lane, sel)
        cumbef = jnp.where(hit, run + s_below, cumbef)
        found = jnp.logical_or(found, hit)
        return run + tot, found, sel, cumbef

    _, _, sel, cumbef = lax.fori_loop(
        0, nvals // L, body,
        (run0, jnp.bool_(False), jnp.int32(0), jnp.int32(0)))
    return sel, cumbef


def _scan_select(hist, gsums, nb, r):
    ngrp = nb // L
    iota = lax.iota(jnp.int32, L)

    def g_body(g, _):
        acc = hist[pl.ds(g * L * L, L)]
        for t in range(1, L):
            acc = acc + hist[pl.ds(g * L * L + t * L, L)]
        s = jnp.sum(acc)
        plsc.store_scatter(gsums, [jnp.full((L,), g, jnp.int32)],
                           jnp.full((L,), s, jnp.int32), mask=iota == 0)
        return 0

    lax.fori_loop(0, ngrp, g_body, 0)
    g_sel, g_cumbef = _scan_vec(gsums, ngrp, r, jnp.int32(0))

    def t_body(t, carry):
        run, found, sel, cumbef = carry
        s = jnp.sum(hist[pl.ds((g_sel * L + t) * L, L)])
        hit = jnp.logical_and(jnp.logical_not(found), (run + s) >= r)
        sel = jnp.where(hit, g_sel * L + t, sel)
        cumbef = jnp.where(hit, run, cumbef)
        found = jnp.logical_or(found, hit)
        return run + s, found, sel, cumbef

    _, _, sel, cumbef = lax.fori_loop(
        0, L, t_body,
        (g_cumbef, jnp.bool_(False), jnp.int32(0), jnp.int32(0)))
    return sel, cumbef


def _weighted_below(hist, prefix2, sel3):
    base = lax.shift_left(prefix2, jnp.int32(10))

    def body(d, acc):
        v = hist[pl.ds(d * L, L)].astype(jnp.float32)
        val = _untransform(jnp.full((L,), jnp.bitwise_or(base, d), jnp.int32))
        return acc + jnp.where(d < sel3, v * val, 0.0)

    acc = lax.fori_loop(0, NB3, body, jnp.zeros((L,), jnp.float32))
    return jnp.sum(acc)


def _sc_body(tp_hbm, ty_hbm, out_hbm,
             pbuf0, ybuf0, pbuf1, ybuf1, hist_p, hist_y, merged,
             outbuf, sem_p0, sem_y0, sem_p1, sem_y1):
    wid = lax.axis_index("s") * 2 + lax.axis_index("c")
    iota = lax.iota(jnp.int32, L)
    zeros_f = jnp.zeros((L,), jnp.float32)
    ones_i = jnp.ones((L,), jnp.int32)

    def didx(d):
        return lax.shift_left(d, jnp.full_like(d, 4)) + iota
    slots = ((pbuf0, ybuf0, sem_p0, sem_y0), (pbuf1, ybuf1, sem_p1, sem_y1))

    def prefetch(c, s):
        bp, by, sp, sy = slots[s]
        rb = c * CHUNK_ROWS
        pltpu.async_copy(tp_hbm.at[wid, pl.ds(rb, CHUNK_ROWS)], bp, sp)
        pltpu.async_copy(ty_hbm.at[wid, pl.ds(rb, CHUNK_ROWS)], by, sy)

    def wait_slot(c, s):
        bp, by, sp, sy = slots[s]
        rb = c * CHUNK_ROWS
        pltpu.make_async_copy(tp_hbm.at[wid, pl.ds(rb, CHUNK_ROWS)], bp,
                              sp).wait()
        pltpu.make_async_copy(ty_hbm.at[wid, pl.ds(rb, CHUNK_ROWS)], by,
                              sy).wait()

    def for_each_vec(s, fn, carry, unroll=8):
        bp, by = slots[s][0], slots[s][1]
        span = unroll * L

        def rowloop(i, car):
            def grp(j, car2):
                tps = [bp[i, pl.ds(j * span + o * L, L)]
                       for o in range(unroll)]
                tys = [by[i, pl.ds(j * span + o * L, L)]
                       for o in range(unroll)]
                return fn(tps, tys, car2)

            return lax.fori_loop(0, W // span, grp, car)

        return lax.fori_loop(0, CHUNK_ROWS, rowloop, carry)

    def run_pass(fn, carry):
        prefetch(0, 0)

        def pair(i, car):
            c0 = 2 * i
            c1 = c0 + 1
            prefetch(c1, 1)
            wait_slot(c0, 0)
            car = for_each_vec(0, fn, car)
            prefetch((c1 + 1) & (NCHUNKS - 1), 0)
            wait_slot(c1, 1)
            car = for_each_vec(1, fn, car)
            return car

        carry = lax.fori_loop(0, NCHUNKS // 2, pair, carry)
        wait_slot(0, 0)
        return carry

    _zero(hist_p, NB1 * L)
    _zero(hist_y, NB1 * L)

    def p1_vec(tps, tys, cacc):
        valids = [tp != -1 for tp in tps]
        ips = [didx(_lsr(tp, 21)) for tp in tps]
        iys = [didx(_lsr(ty, 21)) for ty in tys]
        for v in valids:
            cacc += jnp.where(v, 1.0, 0.0)
        for ip, v in zip(ips, valids):
            plsc.addupdate_scatter(hist_p, [ip], ones_i, mask=v)
        for iy, v in zip(iys, valids):
            plsc.addupdate_scatter(hist_y, [iy], ones_i, mask=v)
        return cacc

    cacc = run_pass(p1_vec, zeros_f)
    cnt_f = jnp.sum(cacc)
    cnt_i = cnt_f.astype(jnp.int32)
    k = lax.shift_right_arithmetic(jnp.maximum(cnt_i, 1) - 1, jnp.int32(1)) + 1

    sel1_p, cb1_p = _scan_select(hist_p, merged, NB1, k)
    sel1_y, cb1_y = _scan_select(hist_y, merged, NB1, k)
    r_p = k - cb1_p
    r_y = k - cb1_y

    _zero(hist_p, NB1 * L)
    _zero(hist_y, NB1 * L)
    mask11 = jnp.full((L,), 0x7FF, jnp.int32)

    def p2_vec(tps, tys, car):
        wps = [_lsr(tp, 21) == sel1_p for tp in tps]
        wys = [_lsr(ty, 21) == sel1_y for ty in tys]
        ips = [didx(jnp.bitwise_and(_lsr(tp, 10), mask11)) for tp in tps]
        iys = [didx(jnp.bitwise_and(_lsr(ty, 10), mask11)) for ty in tys]
        for ip, w in zip(ips, wps):
            plsc.addupdate_scatter(hist_p, [ip], ones_i, mask=w)
        for iy, w in zip(iys, wys):
            plsc.addupdate_scatter(hist_y, [iy], ones_i, mask=w)
        return car

    run_pass(p2_vec, jnp.int32(0))

    sel2_p, cb2_p = _scan_select(hist_p, merged, NB1, r_p)
    sel2_y, cb2_y = _scan_select(hist_y, merged, NB1, r_y)
    pref2_p = jnp.bitwise_or(lax.shift_left(sel1_p, jnp.int32(11)), sel2_p)
    pref2_y = jnp.bitwise_or(lax.shift_left(sel1_y, jnp.int32(11)), sel2_y)
    r_p = r_p - cb2_p
    r_y = r_y - cb2_y

    _zero(hist_p, NB3 * L)
    _zero(hist_y, NB3 * L)
    mask10 = jnp.full((L,), 0x3FF, jnp.int32)

    def p3_vec(tps, tys, car):
        ap, ay = car
        hps = [_lsr(tp, 10) for tp in tps]
        hys = [_lsr(ty, 10) for ty in tys]
        for tp, hp in zip(tps, hps):
            ap += jnp.where(hp < pref2_p, _untransform(tp), 0.0)
        for ty, hy in zip(tys, hys):
            ay += jnp.where(hy < pref2_y, _untransform(ty), 0.0)
        for tp, hp in zip(tps, hps):
            plsc.addupdate_scatter(hist_p, [didx(jnp.bitwise_and(tp, mask10))],
                                   ones_i, mask=hp == pref2_p)
        for ty, hy in zip(tys, hys):
            plsc.addupdate_scatter(hist_y, [didx(jnp.bitwise_and(ty, mask10))],
                                   ones_i, mask=hy == pref2_y)
        return ap, ay

    sa_p, sa_y = run_pass(p3_vec, (zeros_f, zeros_f))
    S_A_p = jnp.sum(sa_p)
    S_A_y = jnp.sum(sa_y)

    sel3_p, cb3_p = _scan_select(hist_p, merged, NB3, r_p)
    w_p = _weighted_below(hist_p, pref2_p, sel3_p)
    sel3_y, cb3_y = _scan_select(hist_y, merged, NB3, r_y)
    w_y = _weighted_below(hist_y, pref2_y, sel3_y)

    pref3_p = jnp.bitwise_or(lax.shift_left(pref2_p, jnp.int32(10)), sel3_p)
    pref3_y = jnp.bitwise_or(lax.shift_left(pref2_y, jnp.int32(10)), sel3_y)
    med_p = _untransform(jnp.full((L,), pref3_p, jnp.int32))
    med_y = _untransform(jnp.full((L,), pref3_y, jnp.int32))
    stats = jnp.where(iota == 0, med_p, zeros_f)
    stats = jnp.where(iota == 1, med_y, stats)
    stats = jnp.where(iota == 2, S_A_p + w_p, stats)
    stats = jnp.where(iota == 3, S_A_y + w_y, stats)
    n_lt_p = jnp.full((L,), cb1_p + cb2_p + cb3_p, jnp.int32)
    n_lt_y = jnp.full((L,), cb1_y + cb2_y + cb3_y, jnp.int32)
    stats = jnp.where(iota == 4, n_lt_p.astype(jnp.float32), stats)
    stats = jnp.where(iota == 5, n_lt_y.astype(jnp.float32), stats)
    stats = jnp.where(iota == 6, cnt_f, stats)
    outbuf[...] = stats
    pltpu.sync_copy(outbuf, out_hbm.at[wid])


_sc_kernel = pl.kernel(
    _sc_body,
    out_type=jax.ShapeDtypeStruct((R_IMGS, L), jnp.float32),
    mesh=_mesh,
    compiler_params=pltpu.CompilerParams(needs_layout_passes=False),
    scratch_types=[
        pltpu.VMEM((CHUNK_ROWS, W), jnp.int32),
        pltpu.VMEM((CHUNK_ROWS, W), jnp.int32),
        pltpu.VMEM((CHUNK_ROWS, W), jnp.int32),
        pltpu.VMEM((CHUNK_ROWS, W), jnp.int32),
        pltpu.VMEM((NB1 * L,), jnp.int32),
        pltpu.VMEM((NB1 * L,), jnp.int32),
        pltpu.VMEM((NB1,), jnp.int32),
        pltpu.VMEM((L,), jnp.float32),
        pltpu.SemaphoreType.DMA,
        pltpu.SemaphoreType.DMA,
        pltpu.SemaphoreType.DMA,
        pltpu.SemaphoreType.DMA,
    ],
)


def _tc_prep_body(p_ref, y_ref, m_ref, tp_ref, ty_ref, mom_ref):
    p = p_ref[0]
    yv = y_ref[0]
    m = m_ref[0].astype(jnp.float32)
    valid = m != 0.0

    def transform(x):
        u = lax.bitcast_convert_type(x, jnp.int32)
        t = jnp.where(u < 0, jnp.bitwise_xor(u, jnp.int32(-1)),
                      jnp.bitwise_or(u, MIN32))
        return jnp.where(valid, t, jnp.int32(-1))

    tp_ref[0] = transform(p)
    ty_ref[0] = transform(yv)
    pm = p * m
    ym = yv * m
    mom_ref[0, 0, :] = jnp.sum(pm, axis=-1)
    mom_ref[0, 1, :] = jnp.sum(ym, axis=-1)
    mom_ref[0, 2, :] = jnp.sum(pm * p, axis=-1)
    mom_ref[0, 3, :] = jnp.sum(ym * yv, axis=-1)
    mom_ref[0, 4, :] = jnp.sum(pm * yv, axis=-1)
    mom_ref[0, 5, :] = jnp.sum(m, axis=-1)
    mom_ref[0, 6, :] = jnp.zeros((W,), jnp.float32)
    mom_ref[0, 7, :] = jnp.zeros((W,), jnp.float32)


def _tc_combine_body(mom_ref, st_ref, o_ref):
    st = st_ref[...]
    med_p = st[:, 0:1]
    med_y = st[:, 1:2]
    SL_p = st[:, 2:3]
    SL_y = st[:, 3:4]
    n_lt_p = st[:, 4:5]
    n_lt_y = st[:, 5:6]
    cnt = st[:, 6:7]
    has = cnt > 0.0
    med_p = jnp.where(has, med_p, 0.0)
    med_y = jnp.where(has, med_y, 0.0)
    S_p = jnp.sum(mom_ref[:, 0, :], axis=-1, keepdims=True)
    S_y = jnp.sum(mom_ref[:, 1, :], axis=-1, keepdims=True)
    safe_cnt = jnp.maximum(cnt, 1.0)
    sum_abs_p = S_p - 2.0 * SL_p + med_p * (2.0 * n_lt_p - cnt)
    sum_abs_y = S_y - 2.0 * SL_y + med_y * (2.0 * n_lt_y - cnt)
    sc_p = jnp.where(has, sum_abs_p / safe_cnt + EPS, EPS)
    sc_y = jnp.where(has, sum_abs_y / safe_cnt + EPS, EPS)
    a = 1.0 / sc_p
    b = -1.0 / sc_y
    cc = -med_p * a - med_y * b
    rho = (a * a) * mom_ref[:, 2, :] + (b * b) * mom_ref[:, 3, :] \
        + (2.0 * a * b) * mom_ref[:, 4, :] + (2.0 * a * cc) * mom_ref[:, 0, :] \
        + (2.0 * b * cc) * mom_ref[:, 1, :] + (cc * cc) * mom_ref[:, 5, :]
    per_row = rho / jnp.maximum(mom_ref[:, 5, :], 1.0)
    val = jnp.sum(per_row) * (1.0 / (R_IMGS * ROWS))
    o_ref[...] = jnp.broadcast_to(val, (1, 1))


@jax.jit
def kernel(pred, y, masks_squeezed):
    if pred.ndim == 5 and pred.shape[2] == 1:
        pred = jnp.squeeze(pred, axis=2)
    if y.ndim == 5 and y.shape[2] == 1:
        y = jnp.squeeze(y, axis=2)
    p = pred.reshape(R_IMGS, ROWS, W)
    q = y.reshape(R_IMGS, ROWS, W)
    m = masks_squeezed.reshape(R_IMGS, ROWS, W).astype(jnp.float32)
    img_spec = pl.BlockSpec((1, ROWS, W), lambda i: (i, 0, 0))
    tp, ty, mom = pl.pallas_call(
        _tc_prep_body,
        grid=(R_IMGS,),
        in_specs=[img_spec, img_spec, img_spec],
        out_specs=[img_spec, img_spec,
                   pl.BlockSpec((1, 8, W), lambda i: (i, 0, 0))],
        out_shape=[
            jax.ShapeDtypeStruct((R_IMGS, ROWS, W), jnp.int32),
            jax.ShapeDtypeStruct((R_IMGS, ROWS, W), jnp.int32),
            jax.ShapeDtypeStruct((R_IMGS, 8, W), jnp.float32),
        ],
    )(p, q, m)
    stats = _sc_kernel(tp, ty)
    out = pl.pallas_call(
        _tc_combine_body,
        out_shape=jax.ShapeDtypeStruct((1, 1), jnp.float32),
    )(mom, stats)
    return out[0, 0]
